# Initial kernel scaffold; baseline (speedup 1.0000x reference)
import jax
import jax.numpy as jnp
from jax.experimental import pallas as pl


def kernel(x, weights, edge_index, W, B):
    raise NotImplementedError("write your pallas kernel here")


# speedup vs baseline: 2.1299x; 2.1299x over previous
import functools

import jax
import jax.numpy as jnp
from jax.experimental import pallas as pl
from jax.experimental.pallas import tpu as pltpu

_N, _D, _E = 10000, 128, 320000
_M = 2000
_NB = _N // _M


def _mix_body(w_ref, x_ref, ns_ref, gs_ref, rsd_ref, inv_ref, W_ref, B_ref, o_ref):
    b = pl.program_id(0)
    ns = jnp.concatenate([ns_ref[0, 0], ns_ref[0, 1]], axis=-1)
    gs = jnp.concatenate([gs_ref[0, 0], gs_ref[0, 1]], axis=-1)
    rsd = rsd_ref[0, 0]
    inv = inv_ref[0, 0]
    x = x_ref[...]
    aggs = (rsd * gs, inv * ns, x + ns)
    acc = None
    for o in range(3):
        h = jnp.dot(aggs[o], W_ref[0, o], preferred_element_type=jnp.float32)
        h = h + B_ref[0, o:o + 1, :]
        e = jnp.where(h > 0, h, jnp.exp(jnp.minimum(h, 0.0)) - 1.0)
        term = w_ref[b, o] * e
        acc = term if acc is None else acc + term
    o_ref[0] = acc


def _mix(weights, x, ns_r, gs_r, rsd, inv, W, B):
    return pl.pallas_call(
        _mix_body,
        grid=(3, _NB),
        in_specs=[
            pl.BlockSpec(memory_space=pltpu.SMEM),
            pl.BlockSpec((_M, _D), lambda b, n: (n, 0)),
            pl.BlockSpec((1, 2, _M, 64), lambda b, n: (b, 0, n, 0)),
            pl.BlockSpec((1, 2, _M, 64), lambda b, n: (b, 0, n, 0)),
            pl.BlockSpec((1, 1, _M, 1), lambda b, n: (b, n, 0, 0)),
            pl.BlockSpec((1, 1, _M, 1), lambda b, n: (b, n, 0, 0)),
            pl.BlockSpec((1, 3, _D, _D), lambda b, n: (b, 0, 0, 0)),
            pl.BlockSpec((1, 3, _D), lambda b, n: (b, 0, 0)),
        ],
        out_specs=pl.BlockSpec((1, _M, _D), lambda b, n: (b, n, 0)),
        out_shape=jax.ShapeDtypeStruct((3, _N, _D), jnp.float32),
    )(weights, x, ns_r, gs_r, rsd, inv, W, B)


def kernel(x, weights, edge_index, W, B):
    ones = jnp.ones((_E,), jnp.float32)
    ns_l, gs_l, rsd_l, inv_l = [], [], [], []
    for b in range(3):
        src = edge_index[b, 0]
        dst = edge_index[b, 1]
        deg_s = jax.ops.segment_sum(ones, src, num_segments=_N)
        deg_d = jax.ops.segment_sum(ones, dst, num_segments=_N)
        rs_s = jax.lax.rsqrt(deg_s + 1.0)
        rs_d = jax.lax.rsqrt(deg_d + 1.0)
        inv = 1.0 / jnp.maximum(deg_d, 1.0)
        x2 = x * rs_s[:, None]
        nsum = jax.ops.segment_sum(x[src], dst, num_segments=_N)
        gsum = jax.ops.segment_sum(x2[src], dst, num_segments=_N)
        ns_l.append(nsum)
        gs_l.append(gsum)
        rsd_l.append(rs_d)
        inv_l.append(inv)
    nsum = jnp.stack(ns_l)
    gsum = jnp.stack(gs_l)
    rsd = jnp.stack(rsd_l)
    inv = jnp.stack(inv_l)
    ns_r = nsum.reshape(3, _N, 2, 64).transpose(0, 2, 1, 3)
    gs_r = gsum.reshape(3, _N, 2, 64).transpose(0, 2, 1, 3)
    rsd4 = rsd.reshape(3, _NB, _M, 1)
    inv4 = inv.reshape(3, _NB, _M, 1)
    return _mix(weights, x, ns_r, gs_r, rsd4, inv4, W, B)


# speedup vs baseline: 8.1933x; 3.8468x over previous
import functools

import jax
import jax.numpy as jnp
from jax import lax
from jax.experimental import pallas as pl
from jax.experimental.pallas import tpu as pltpu
from jax.experimental.pallas import tpu_sc as plsc

_N, _D, _E = 10000, 128, 320000
_NPAD = 10240
_M = 2048
_NB = _NPAD // _M
_NT = 32
_EH = _E // _NT
_ET = _E // 16
_CK = 80
_NCH = _ET // _CK


def _sc_mesh():
    return plsc.VectorSubcoreMesh(core_axis_name="c", subcore_axis_name="s")


def _hist_body(ei, deg_out, hist_v, idx_v, red_v, out_v, shared_h):
    c = lax.axis_index("c")
    s = lax.axis_index("s")
    wid = s * 2 + c
    ones16 = jnp.full((16,), 1.0, jnp.float32)
    zeros16 = jnp.zeros((16,), jnp.float32)
    for h in range(6):
        def zero_body(i):
            hist_v[pl.ds(i * 16, 16)] = zeros16
        lax.fori_loop(0, _NPAD // 16, lambda i, _: (zero_body(i), _)[1], None)
        pltpu.sync_copy(ei.at[h, wid], idx_v)

        def acc_body(j):
            idx16 = idx_v[pl.ds(j * 16, 16)]
            plsc.addupdate_scatter(hist_v, [idx16], ones16)
        lax.fori_loop(0, _EH // 16, lambda j, _: (acc_body(j), _)[1], None)

        pltpu.sync_copy(hist_v, shared_h.at[s])
        plsc.subcore_barrier()
        for r in range(16):
            pltpu.sync_copy(shared_h.at[r, pl.ds(s * 640, 640)], red_v.at[r])

        def red_body(j):
            acc = red_v[0, pl.ds(j * 16, 16)]
            for r in range(1, 16):
                acc = acc + red_v[r, pl.ds(j * 16, 16)]
            out_v[pl.ds(j * 16, 16)] = acc
        lax.fori_loop(0, 40, lambda j, _: (red_body(j), _)[1], None)
        pltpu.sync_copy(out_v, deg_out.at[c, h, pl.ds(s * 640, 640)])
        plsc.subcore_barrier()


def _histograms(ei_h):
    k = pl.kernel(
        _hist_body,
        out_type=jax.ShapeDtypeStruct((2, 6, _NPAD), jnp.float32),
        mesh=_sc_mesh(),
        compiler_params=pltpu.CompilerParams(needs_layout_passes=False),
        scratch_types=[
            pltpu.VMEM((_NPAD,), jnp.float32),
            pltpu.VMEM((_EH,), jnp.int32),
            pltpu.VMEM((16, 640), jnp.float32),
            pltpu.VMEM((640,), jnp.float32),
            pltpu.VMEM_SHARED((16, _NPAD), jnp.float32),
        ],
    )
    return k(ei_h)


def _prep_body(dp_ref, sc3_ref):
    deg = dp_ref[0] + dp_ref[1]
    deg_s = jnp.concatenate([deg[0:1], deg[2:3], deg[4:5]], axis=0)
    deg_d = jnp.concatenate([deg[1:2], deg[3:4], deg[5:6]], axis=0)
    sc3_ref[0] = lax.rsqrt(deg_d + 1.0)
    sc3_ref[1] = 1.0 / jnp.maximum(deg_d, 1.0)
    sc3_ref[2] = lax.rsqrt(deg_s + 1.0)


def _prep(deg_part):
    return pl.pallas_call(
        _prep_body,
        out_shape=jax.ShapeDtypeStruct((3, 3, _NPAD), jnp.float32),
    )(deg_part)


def _table_body(x_ref, rs_ref, o_ref):
    t = pl.program_id(0)
    xb = x_ref[0]
    rs = rs_ref[0]
    scale = jnp.where(t == 0, jnp.ones_like(rs), rs)
    o_ref[0, 0] = xb * scale


def _table(xh, rs_s3):
    return pl.pallas_call(
        _table_body,
        grid=(4, 2),
        in_specs=[
            pl.BlockSpec((1, _NPAD, 64), lambda t, c: (c, 0, 0)),
            pl.BlockSpec((1, _NPAD, 1), lambda t, c: (jnp.maximum(t - 1, 0), 0, 0)),
        ],
        out_specs=pl.BlockSpec((1, 1, _NPAD, 64), lambda t, c: (t, c, 0, 0)),
        out_shape=jax.ShapeDtypeStruct((4, 2, _NPAD, 64), jnp.float32),
    )(xh, rs_s3)


def _agg_body(xt, srcr, dstr, acc_out, src_v, adj_v, dst_v, rows_v, zero_v,
              acc_sh, sem):
    c = lax.axis_index("c")
    s = lax.axis_index("s")
    zeros16 = jnp.zeros((16,), jnp.float32)

    def zz_body(i):
        for k in range(64 // 16):
            zero_v[i, pl.ds(k * 16, 16)] = zeros16
    lax.fori_loop(0, _CK, lambda i, _: (zz_body(i), _)[1], None)

    for b in range(3):
        pltpu.sync_copy(srcr.at[b, s], src_v)
        pltpu.sync_copy(dstr.at[b, s], dst_v)
        for p, tslot in ((0, 0), (1, 1 + b)):
            for k in range(8):
                pltpu.sync_copy(zero_v, acc_sh.at[pl.ds(s * 640 + k * _CK, _CK)])
            off = (tslot * 2 + c) * _NPAD

            def adj_body(ch):
                for k in range(_CK // 16):
                    v = src_v[ch, pl.ds(k * 16, 16)]
                    adj_v[ch, pl.ds(k * 16, 16)] = v + off
            lax.fori_loop(0, _NCH, lambda ch, _: (adj_body(ch), _)[1], None)
            plsc.subcore_barrier()

            def chunk_body(ch):
                pltpu.async_copy(xt.at[adj_v.at[ch]], rows_v, sem).wait()
                pltpu.sync_copy(rows_v, acc_sh.at[dst_v.at[ch]], add=True)
            lax.fori_loop(0, _NCH, lambda ch, _: (chunk_body(ch), _)[1], None)
            plsc.subcore_barrier()

            for k in range(8):
                sl = pl.ds(s * 640 + k * _CK, _CK)
                pltpu.sync_copy(acc_sh.at[sl], acc_out.at[b, p, c, sl])
            plsc.subcore_barrier()


def _aggregate(xt_flat, srcr, dstr):
    k = pl.kernel(
        _agg_body,
        out_type=jax.ShapeDtypeStruct((3, 2, 2, _NPAD, 64), jnp.float32),
        mesh=_sc_mesh(),
        compiler_params=pltpu.CompilerParams(
            needs_layout_passes=False, use_tc_tiling_on_sc=False),
        scratch_types=[
            pltpu.VMEM((_NCH, _CK), jnp.int32),
            pltpu.VMEM((_NCH, _CK), jnp.int32),
            pltpu.VMEM((_NCH, _CK), jnp.int32),
            pltpu.VMEM((_CK, 64), jnp.float32),
            pltpu.VMEM((_CK, 64), jnp.float32),
            pltpu.VMEM_SHARED((_NPAD, 64), jnp.float32),
            pltpu.SemaphoreType.DMA,
        ],
    )
    return k(xt_flat, srcr, dstr)


def _mix_body(w_ref, x_ref, ns_ref, gs_ref, rsd_ref, inv_ref, W_ref, B_ref, o_ref):
    b = pl.program_id(0)
    ns = jnp.concatenate([ns_ref[0, 0, 0], ns_ref[0, 0, 1]], axis=-1)
    gs = jnp.concatenate([gs_ref[0, 0, 0], gs_ref[0, 0, 1]], axis=-1)
    rsd = rsd_ref[0, 0]
    inv = inv_ref[0, 0]
    x = x_ref[...]
    aggs = (rsd * gs, inv * ns, x + ns)
    acc = None
    for o in range(3):
        h = jnp.dot(aggs[o], W_ref[0, o], preferred_element_type=jnp.float32)
        h = h + B_ref[0, o:o + 1, :]
        e = jnp.where(h > 0, h, jnp.exp(jnp.minimum(h, 0.0)) - 1.0)
        term = w_ref[b, o] * e
        acc = term if acc is None else acc + term
    o_ref[0] = acc


def _mix(weights, x_pad, acc_r, rsd4, inv4, W, B):
    return pl.pallas_call(
        _mix_body,
        grid=(3, _NB),
        in_specs=[
            pl.BlockSpec(memory_space=pltpu.SMEM),
            pl.BlockSpec((_M, _D), lambda b, n: (n, 0)),
            pl.BlockSpec((1, 1, 2, _M, 64), lambda b, n: (b, 0, 0, n, 0)),
            pl.BlockSpec((1, 1, 2, _M, 64), lambda b, n: (b, 1, 0, n, 0)),
            pl.BlockSpec((1, 1, _M, 1), lambda b, n: (b, n, 0, 0)),
            pl.BlockSpec((1, 1, _M, 1), lambda b, n: (b, n, 0, 0)),
            pl.BlockSpec((1, 3, _D, _D), lambda b, n: (b, 0, 0, 0)),
            pl.BlockSpec((1, 3, _D), lambda b, n: (b, 0, 0)),
        ],
        out_specs=pl.BlockSpec((1, _M, _D), lambda b, n: (b, n, 0)),
        out_shape=jax.ShapeDtypeStruct((3, _NPAD, _D), jnp.float32),
    )(weights, x_pad, acc_r, acc_r, rsd4, inv4, W, B)


def kernel(x, weights, edge_index, W, B):
    ei_h = edge_index.reshape(6, _NT, _EH)
    srcr = edge_index[:, 0, :].reshape(3, 16, _NCH, _CK)
    dstr = edge_index[:, 1, :].reshape(3, 16, _NCH, _CK)
    x_pad = jnp.pad(x, ((0, _NPAD - _N), (0, 0)))

    deg_part = _histograms(ei_h)
    sc3 = _prep(deg_part)
    rs_s3 = sc3[2].reshape(3, _NPAD, 1)
    xh = x_pad.reshape(_NPAD, 2, 64).transpose(1, 0, 2)
    xt2 = _table(xh, rs_s3)
    acc_r = _aggregate(xt2.reshape(8 * _NPAD, 64), srcr, dstr)
    rsd4 = sc3[0].reshape(3, _NB, _M, 1)
    inv4 = sc3[1].reshape(3, _NB, _M, 1)
    out = _mix(weights, x_pad, acc_r, rsd4, inv4, W, B)
    return out[:, :_N, :]


# speedup vs baseline: 14.8893x; 1.8173x over previous
import functools

import jax
import jax.numpy as jnp
from jax import lax
from jax.experimental import pallas as pl
from jax.experimental.pallas import tpu as pltpu
from jax.experimental.pallas import tpu_sc as plsc

_N, _D, _E = 10000, 128, 320000
_NPAD = 10240
_M = 2048
_NB = _NPAD // _M
_NT = 32
_EH = _E // _NT
_ET = _E // 16
_CK = 80
_NCH = _ET // _CK


def _sc_mesh():
    return plsc.VectorSubcoreMesh(core_axis_name="c", subcore_axis_name="s")


def _hist_body(ei, deg_out, hist_v, idx_v, red_v, out_v, shared_h):
    c = lax.axis_index("c")
    s = lax.axis_index("s")
    wid = s * 2 + c
    ones16 = jnp.full((16,), 1.0, jnp.float32)
    zeros16 = jnp.zeros((16,), jnp.float32)
    for h in range(6):
        def zero_body(i):
            hist_v[pl.ds(i * 16, 16)] = zeros16
        lax.fori_loop(0, _NPAD // 16, lambda i, _: (zero_body(i), _)[1], None)
        pltpu.sync_copy(ei.at[h, wid], idx_v)

        def acc_body(j):
            idx16 = idx_v[pl.ds(j * 16, 16)]
            plsc.addupdate_scatter(hist_v, [idx16], ones16)
        lax.fori_loop(0, _EH // 16, lambda j, _: (acc_body(j), _)[1], None)

        pltpu.sync_copy(hist_v, shared_h.at[s])
        plsc.subcore_barrier()
        for r in range(16):
            pltpu.sync_copy(shared_h.at[r, pl.ds(s * 640, 640)], red_v.at[r])

        def red_body(j):
            acc = red_v[0, pl.ds(j * 16, 16)]
            for r in range(1, 16):
                acc = acc + red_v[r, pl.ds(j * 16, 16)]
            out_v[pl.ds(j * 16, 16)] = acc
        lax.fori_loop(0, 40, lambda j, _: (red_body(j), _)[1], None)
        pltpu.sync_copy(out_v, deg_out.at[c, h, pl.ds(s * 640, 640)])
        plsc.subcore_barrier()


def _histograms(ei_h):
    k = pl.kernel(
        _hist_body,
        out_type=jax.ShapeDtypeStruct((2, 6, _NPAD), jnp.float32),
        mesh=_sc_mesh(),
        compiler_params=pltpu.CompilerParams(needs_layout_passes=False),
        scratch_types=[
            pltpu.VMEM((_NPAD,), jnp.float32),
            pltpu.VMEM((_EH,), jnp.int32),
            pltpu.VMEM((16, 640), jnp.float32),
            pltpu.VMEM((640,), jnp.float32),
            pltpu.VMEM_SHARED((16, _NPAD), jnp.float32),
        ],
    )
    return k(ei_h)


def _prep_body(dp_ref, sc3_ref):
    deg = dp_ref[0] + dp_ref[1]
    deg_s = jnp.concatenate([deg[0:1], deg[2:3], deg[4:5]], axis=0)
    deg_d = jnp.concatenate([deg[1:2], deg[3:4], deg[5:6]], axis=0)
    sc3_ref[0] = lax.rsqrt(deg_d + 1.0)
    sc3_ref[1] = 1.0 / jnp.maximum(deg_d, 1.0)
    sc3_ref[2] = lax.rsqrt(deg_s + 1.0)


def _prep(deg_part):
    return pl.pallas_call(
        _prep_body,
        out_shape=jax.ShapeDtypeStruct((3, 3, _NPAD), jnp.float32),
    )(deg_part)


def _table_body(x_ref, rs_ref, o_ref):
    t = pl.program_id(0)
    xb = x_ref[0]
    rs = rs_ref[0]
    scale = jnp.where(t == 0, jnp.ones_like(rs), rs)
    o_ref[0, 0] = xb * scale


def _table(xh, rs_s3):
    return pl.pallas_call(
        _table_body,
        grid=(4, 2),
        in_specs=[
            pl.BlockSpec((1, _NPAD, 64), lambda t, c: (c, 0, 0)),
            pl.BlockSpec((1, _NPAD, 1), lambda t, c: (jnp.maximum(t - 1, 0), 0, 0)),
        ],
        out_specs=pl.BlockSpec((1, 1, _NPAD, 64), lambda t, c: (t, c, 0, 0)),
        out_shape=jax.ShapeDtypeStruct((4, 2, _NPAD, 64), jnp.float32),
    )(xh, rs_s3)


_NBUF = 5
_LOOK = 2


def _agg_body(xt, srcr, dstr, acc_out, adj_v, dst_v, rows_v, zero_v,
              acc_sh, gs0, gs1, gs2, gs3, gs4, ssem):
    c = lax.axis_index("c")
    s = lax.axis_index("s")
    zeros16 = jnp.zeros((16,), jnp.float32)
    gsems = (gs0, gs1, gs2, gs3, gs4)

    def zz_body(i):
        for k in range(64 // 16):
            zero_v[i, pl.ds(k * 16, 16)] = zeros16
    lax.fori_loop(0, _CK, lambda i, _: (zz_body(i), _)[1], None)

    def start_g(g, bi):
        pltpu.async_copy(xt.at[adj_v.at[g]], rows_v.at[bi], gsems[bi])

    def wait_g(g, bi):
        pltpu.make_async_copy(xt.at[adj_v.at[g]], rows_v.at[bi], gsems[bi]).wait()

    def start_s(g, bi):
        pltpu.async_copy(rows_v.at[bi], acc_sh.at[dst_v.at[g]], ssem, add=True)

    def drain_s(g, bi):
        pltpu.make_async_copy(rows_v.at[bi], acc_sh.at[dst_v.at[g]], ssem).wait()

    def step(g, bi, drain, ahead):
        wait_g(g, bi)
        start_s(g, bi)
        if drain:
            drain_s(g, bi)
        if ahead:
            start_g(g + _LOOK, (bi + _LOOK) % _NBUF)

    for b in range(3):
        pltpu.sync_copy(srcr.at[b, s], adj_v)
        pltpu.sync_copy(dstr.at[b, s], dst_v)
        for p in (0, 1):
            for k in range(8):
                pltpu.sync_copy(zero_v, acc_sh.at[pl.ds(s * 640 + k * _CK, _CK)])
            off = c * _NPAD if p == 0 else (1 + b) * 2 * _NPAD

            def adj_body(ch):
                for k in range(_CK // 16):
                    v = adj_v[ch, pl.ds(k * 16, 16)]
                    adj_v[ch, pl.ds(k * 16, 16)] = v + off
            lax.fori_loop(0, _NCH, lambda ch, _: (adj_body(ch), _)[1], None)
            plsc.subcore_barrier()

            start_g(0, 0)
            start_g(1, 1)
            for bi in range(_NBUF):
                step(bi, bi, drain=(bi >= 3), ahead=True)

            def group(go):
                for bi in range(_NBUF):
                    step(go * _NBUF + bi, bi, True, True)
            lax.fori_loop(1, _NCH // _NBUF - 1,
                          lambda go, _: (group(go), _)[1], None)

            for bi in range(_NBUF):
                g = _NCH - _NBUF + bi
                step(g, bi, True, ahead=(g + _LOOK < _NCH))
            for _ in range(3):
                drain_s(_NCH - 1, _NBUF - 1)
            plsc.subcore_barrier()

            for k in range(8):
                sl = pl.ds(s * 640 + k * _CK, _CK)
                pltpu.sync_copy(acc_sh.at[sl], acc_out.at[b, p, c, sl])
            plsc.subcore_barrier()


def _aggregate(xt_flat, srcr, dstr):
    k = pl.kernel(
        _agg_body,
        out_type=jax.ShapeDtypeStruct((3, 2, 2, _NPAD, 64), jnp.float32),
        mesh=_sc_mesh(),
        compiler_params=pltpu.CompilerParams(
            needs_layout_passes=False, use_tc_tiling_on_sc=False),
        scratch_types=[
            pltpu.VMEM((_NCH, _CK), jnp.int32),
            pltpu.VMEM((_NCH, _CK), jnp.int32),
            pltpu.VMEM((_NBUF, _CK, 64), jnp.float32),
            pltpu.VMEM((_CK, 64), jnp.float32),
            pltpu.VMEM_SHARED((_NPAD, 64), jnp.float32),
            pltpu.SemaphoreType.DMA,
            pltpu.SemaphoreType.DMA,
            pltpu.SemaphoreType.DMA,
            pltpu.SemaphoreType.DMA,
            pltpu.SemaphoreType.DMA,
            pltpu.SemaphoreType.DMA,
        ],
    )
    return k(xt_flat, srcr, dstr)


def _mix_body(w_ref, x_ref, ns_ref, gs_ref, rsd_ref, inv_ref, W_ref, B_ref, o_ref):
    b = pl.program_id(0)
    ns = jnp.concatenate([ns_ref[0, 0, 0], ns_ref[0, 0, 1]], axis=-1)
    gs = jnp.concatenate([gs_ref[0, 0, 0], gs_ref[0, 0, 1]], axis=-1)
    rsd = rsd_ref[0, 0]
    inv = inv_ref[0, 0]
    x = x_ref[...]
    aggs = (rsd * gs, inv * ns, x + ns)
    acc = None
    for o in range(3):
        h = jnp.dot(aggs[o], W_ref[0, o], preferred_element_type=jnp.float32)
        h = h + B_ref[0, o:o + 1, :]
        e = jnp.where(h > 0, h, jnp.exp(jnp.minimum(h, 0.0)) - 1.0)
        term = w_ref[b, o] * e
        acc = term if acc is None else acc + term
    o_ref[0] = acc


def _mix(weights, x_pad, acc_r, rsd4, inv4, W, B):
    return pl.pallas_call(
        _mix_body,
        grid=(3, _NB),
        in_specs=[
            pl.BlockSpec(memory_space=pltpu.SMEM),
            pl.BlockSpec((_M, _D), lambda b, n: (n, 0)),
            pl.BlockSpec((1, 1, 2, _M, 64), lambda b, n: (b, 0, 0, n, 0)),
            pl.BlockSpec((1, 1, 2, _M, 64), lambda b, n: (b, 1, 0, n, 0)),
            pl.BlockSpec((1, 1, _M, 1), lambda b, n: (b, n, 0, 0)),
            pl.BlockSpec((1, 1, _M, 1), lambda b, n: (b, n, 0, 0)),
            pl.BlockSpec((1, 3, _D, _D), lambda b, n: (b, 0, 0, 0)),
            pl.BlockSpec((1, 3, _D), lambda b, n: (b, 0, 0)),
        ],
        out_specs=pl.BlockSpec((1, _M, _D), lambda b, n: (b, n, 0)),
        out_shape=jax.ShapeDtypeStruct((3, _NPAD, _D), jnp.float32),
    )(weights, x_pad, acc_r, acc_r, rsd4, inv4, W, B)


def kernel(x, weights, edge_index, W, B):
    ei_h = edge_index.reshape(6, _NT, _EH)
    srcr = edge_index[:, 0, :].reshape(3, 16, _NCH, _CK)
    dstr = edge_index[:, 1, :].reshape(3, 16, _NCH, _CK)
    x_pad = jnp.pad(x, ((0, _NPAD - _N), (0, 0)))

    deg_part = _histograms(ei_h)
    sc3 = _prep(deg_part)
    rs_s3 = sc3[2].reshape(3, _NPAD, 1)
    xh = x_pad.reshape(_NPAD, 2, 64).transpose(1, 0, 2)
    xt2 = _table(xh, rs_s3)
    acc_r = _aggregate(xt2.reshape(8 * _NPAD, 64), srcr, dstr)
    rsd4 = sc3[0].reshape(3, _NB, _M, 1)
    inv4 = sc3[1].reshape(3, _NB, _M, 1)
    out = _mix(weights, x_pad, acc_r, rsd4, inv4, W, B)
    return out[:, :_N, :]


# speedup vs baseline: 17.2821x; 1.1607x over previous
import functools

import jax
import jax.numpy as jnp
from jax import lax
from jax.experimental import pallas as pl
from jax.experimental.pallas import tpu as pltpu
from jax.experimental.pallas import tpu_sc as plsc

_N, _D, _E = 10000, 128, 320000
_NPAD = 10240
_M = 2048
_NB = _NPAD // _M
_NT = 32
_EH = _E // _NT
_ET = _E // 16
_CK = 80
_NCH = _ET // _CK


def _sc_mesh():
    return plsc.VectorSubcoreMesh(core_axis_name="c", subcore_axis_name="s")


def _hist_body(ei, deg_out, hist_v, idx_v, red_v, out_v, shared_h):
    c = lax.axis_index("c")
    s = lax.axis_index("s")
    wid = s * 2 + c
    ones16 = jnp.full((16,), 1.0, jnp.float32)
    zeros16 = jnp.zeros((16,), jnp.float32)
    for h in range(6):
        def zero_body(i):
            hist_v[pl.ds(i * 16, 16)] = zeros16
        lax.fori_loop(0, _NPAD // 16, lambda i, _: (zero_body(i), _)[1], None)
        pltpu.sync_copy(ei.at[h, wid], idx_v)

        def acc_body(j):
            idx16 = idx_v[pl.ds(j * 16, 16)]
            plsc.addupdate_scatter(hist_v, [idx16], ones16)
        lax.fori_loop(0, _EH // 16, lambda j, _: (acc_body(j), _)[1], None)

        pltpu.sync_copy(hist_v, shared_h.at[s])
        plsc.subcore_barrier()
        for r in range(16):
            pltpu.sync_copy(shared_h.at[r, pl.ds(s * 640, 640)], red_v.at[r])

        def red_body(j):
            acc = red_v[0, pl.ds(j * 16, 16)]
            for r in range(1, 16):
                acc = acc + red_v[r, pl.ds(j * 16, 16)]
            out_v[pl.ds(j * 16, 16)] = acc
        lax.fori_loop(0, 40, lambda j, _: (red_body(j), _)[1], None)
        pltpu.sync_copy(out_v, deg_out.at[c, h, pl.ds(s * 640, 640)])
        plsc.subcore_barrier()


def _histograms(ei_h):
    k = pl.kernel(
        _hist_body,
        out_type=jax.ShapeDtypeStruct((2, 6, _NPAD), jnp.float32),
        mesh=_sc_mesh(),
        compiler_params=pltpu.CompilerParams(needs_layout_passes=False),
        scratch_types=[
            pltpu.VMEM((_NPAD,), jnp.float32),
            pltpu.VMEM((_EH,), jnp.int32),
            pltpu.VMEM((16, 640), jnp.float32),
            pltpu.VMEM((640,), jnp.float32),
            pltpu.VMEM_SHARED((16, _NPAD), jnp.float32),
        ],
    )
    return k(ei_h)


def _prep_body(dp_ref, sc3_ref):
    deg = dp_ref[0] + dp_ref[1]
    deg_s = jnp.concatenate([deg[0:1], deg[2:3], deg[4:5]], axis=0)
    deg_d = jnp.concatenate([deg[1:2], deg[3:4], deg[5:6]], axis=0)
    sc3_ref[0] = lax.rsqrt(deg_d + 1.0)
    sc3_ref[1] = 1.0 / jnp.maximum(deg_d, 1.0)
    sc3_ref[2] = lax.rsqrt(deg_s + 1.0)


def _prep(deg_part):
    return pl.pallas_call(
        _prep_body,
        out_shape=jax.ShapeDtypeStruct((3, 3, _NPAD), jnp.float32),
    )(deg_part)


def _table_body(x_ref, rs_ref, o_ref):
    t = pl.program_id(0)
    xb = x_ref[0]
    rs = rs_ref[0]
    scale = jnp.where(t == 0, jnp.ones_like(rs), rs)
    o_ref[0, 0] = xb * scale


def _table(xh, rs_s3):
    return pl.pallas_call(
        _table_body,
        grid=(4, 2),
        in_specs=[
            pl.BlockSpec((1, _NPAD, 64), lambda t, c: (c, 0, 0)),
            pl.BlockSpec((1, _NPAD, 1), lambda t, c: (jnp.maximum(t - 1, 0), 0, 0)),
        ],
        out_specs=pl.BlockSpec((1, 1, _NPAD, 64), lambda t, c: (t, c, 0, 0)),
        out_shape=jax.ShapeDtypeStruct((4, 2, _NPAD, 64), jnp.float32),
    )(xh, rs_s3)


_NBUF = 5
_LOOK = 3


def _agg_body(xt, srcr, dstr, acc_out, adj_v, dst_v, rows_v, zero_v,
              acc_sh, gs0, gs1, gs2, gs3, gs4, ssem):
    c = lax.axis_index("c")
    s = lax.axis_index("s")
    zeros16 = jnp.zeros((16,), jnp.float32)
    gsems = (gs0, gs1, gs2, gs3, gs4)

    def zz_body(i):
        for k in range(64 // 16):
            zero_v[i, pl.ds(k * 16, 16)] = zeros16
    lax.fori_loop(0, _CK, lambda i, _: (zz_body(i), _)[1], None)

    def start_g(g, bi):
        pltpu.async_copy(xt.at[adj_v.at[g]], rows_v.at[bi], gsems[bi])

    def wait_g(g, bi):
        pltpu.make_async_copy(xt.at[adj_v.at[g]], rows_v.at[bi], gsems[bi]).wait()

    def start_s(g, bi):
        pltpu.async_copy(rows_v.at[bi], acc_sh.at[dst_v.at[g]], ssem, add=True)

    def drain_s(g, bi):
        pltpu.make_async_copy(rows_v.at[bi], acc_sh.at[dst_v.at[g]], ssem).wait()

    def step(g, bi, drain, ahead):
        wait_g(g, bi)
        start_s(g, bi)
        if drain:
            drain_s(g, bi)
        if ahead:
            start_g(g + _LOOK, (bi + _LOOK) % _NBUF)

    for b in range(3):
        pltpu.sync_copy(srcr.at[b, s], adj_v)
        pltpu.sync_copy(dstr.at[b, s], dst_v)
        for p in (0, 1):
            for k in range(8):
                pltpu.sync_copy(zero_v, acc_sh.at[pl.ds(s * 640 + k * _CK, _CK)])
            off = c * _NPAD if p == 0 else (1 + b) * 2 * _NPAD

            def adj_body(ch):
                for k in range(_CK // 16):
                    v = adj_v[ch, pl.ds(k * 16, 16)]
                    adj_v[ch, pl.ds(k * 16, 16)] = v + off
            lax.fori_loop(0, _NCH, lambda ch, _: (adj_body(ch), _)[1], None)
            plsc.subcore_barrier()

            for j in range(_LOOK):
                start_g(j, j)
            for bi in range(_NBUF):
                step(bi, bi, drain=(bi >= _NBUF - _LOOK), ahead=True)

            def group(go):
                for bi in range(_NBUF):
                    step(go * _NBUF + bi, bi, True, True)
            lax.fori_loop(1, _NCH // _NBUF - 1,
                          lambda go, _: (group(go), _)[1], None)

            for bi in range(_NBUF):
                g = _NCH - _NBUF + bi
                step(g, bi, True, ahead=(g + _LOOK < _NCH))
            for _ in range(_NBUF - _LOOK):
                drain_s(_NCH - 1, _NBUF - 1)
            plsc.subcore_barrier()

            for k in range(8):
                sl = pl.ds(s * 640 + k * _CK, _CK)
                pltpu.sync_copy(acc_sh.at[sl], acc_out.at[b, p, c, sl])
            plsc.subcore_barrier()


def _aggregate(xt_flat, srcr, dstr):
    k = pl.kernel(
        _agg_body,
        out_type=jax.ShapeDtypeStruct((3, 2, 2, _NPAD, 64), jnp.float32),
        mesh=_sc_mesh(),
        compiler_params=pltpu.CompilerParams(
            needs_layout_passes=False, use_tc_tiling_on_sc=False),
        scratch_types=[
            pltpu.VMEM((_NCH, _CK), jnp.int32),
            pltpu.VMEM((_NCH, _CK), jnp.int32),
            pltpu.VMEM((_NBUF, _CK, 64), jnp.float32),
            pltpu.VMEM((_CK, 64), jnp.float32),
            pltpu.VMEM_SHARED((_NPAD, 64), jnp.float32),
            pltpu.SemaphoreType.DMA,
            pltpu.SemaphoreType.DMA,
            pltpu.SemaphoreType.DMA,
            pltpu.SemaphoreType.DMA,
            pltpu.SemaphoreType.DMA,
            pltpu.SemaphoreType.DMA,
        ],
    )
    return k(xt_flat, srcr, dstr)


def _mix_body(w_ref, x_ref, ns_ref, gs_ref, rsd_ref, inv_ref, W_ref, B_ref, o_ref):
    b = pl.program_id(0)
    ns = jnp.concatenate([ns_ref[0, 0, 0], ns_ref[0, 0, 1]], axis=-1)
    gs = jnp.concatenate([gs_ref[0, 0, 0], gs_ref[0, 0, 1]], axis=-1)
    rsd = rsd_ref[0, 0]
    inv = inv_ref[0, 0]
    x = x_ref[...]
    aggs = (rsd * gs, inv * ns, x + ns)
    acc = None
    for o in range(3):
        h = jnp.dot(aggs[o], W_ref[0, o], preferred_element_type=jnp.float32)
        h = h + B_ref[0, o:o + 1, :]
        e = jnp.where(h > 0, h, jnp.exp(jnp.minimum(h, 0.0)) - 1.0)
        term = w_ref[b, o] * e
        acc = term if acc is None else acc + term
    o_ref[0] = acc


def _mix(weights, x_pad, acc_r, rsd4, inv4, W, B):
    return pl.pallas_call(
        _mix_body,
        grid=(3, _NB),
        in_specs=[
            pl.BlockSpec(memory_space=pltpu.SMEM),
            pl.BlockSpec((_M, _D), lambda b, n: (n, 0)),
            pl.BlockSpec((1, 1, 2, _M, 64), lambda b, n: (b, 0, 0, n, 0)),
            pl.BlockSpec((1, 1, 2, _M, 64), lambda b, n: (b, 1, 0, n, 0)),
            pl.BlockSpec((1, 1, _M, 1), lambda b, n: (b, n, 0, 0)),
            pl.BlockSpec((1, 1, _M, 1), lambda b, n: (b, n, 0, 0)),
            pl.BlockSpec((1, 3, _D, _D), lambda b, n: (b, 0, 0, 0)),
            pl.BlockSpec((1, 3, _D), lambda b, n: (b, 0, 0)),
        ],
        out_specs=pl.BlockSpec((1, _M, _D), lambda b, n: (b, n, 0)),
        out_shape=jax.ShapeDtypeStruct((3, _NPAD, _D), jnp.float32),
    )(weights, x_pad, acc_r, acc_r, rsd4, inv4, W, B)


def kernel(x, weights, edge_index, W, B):
    ei_h = edge_index.reshape(6, _NT, _EH)
    srcr = edge_index[:, 0, :].reshape(3, 16, _NCH, _CK)
    dstr = edge_index[:, 1, :].reshape(3, 16, _NCH, _CK)
    x_pad = jnp.pad(x, ((0, _NPAD - _N), (0, 0)))

    deg_part = _histograms(ei_h)
    sc3 = _prep(deg_part)
    rs_s3 = sc3[2].reshape(3, _NPAD, 1)
    xh = x_pad.reshape(_NPAD, 2, 64).transpose(1, 0, 2)
    xt2 = _table(xh, rs_s3)
    acc_r = _aggregate(xt2.reshape(8 * _NPAD, 64), srcr, dstr)
    rsd4 = sc3[0].reshape(3, _NB, _M, 1)
    inv4 = sc3[1].reshape(3, _NB, _M, 1)
    out = _mix(weights, x_pad, acc_r, rsd4, inv4, W, B)
    return out[:, :_N, :]


# speedup vs baseline: 19.0292x; 1.1011x over previous
import functools

import jax
import jax.numpy as jnp
from jax import lax
from jax.experimental import pallas as pl
from jax.experimental.pallas import tpu as pltpu
from jax.experimental.pallas import tpu_sc as plsc

_N, _D, _E = 10000, 128, 320000
_NPAD = 10240
_M = 2048
_NB = _NPAD // _M
_NT = 32
_EH = _E // _NT
_ET = _E // 16
_CK = 80
_NCH = _ET // _CK


def _sc_mesh():
    return plsc.VectorSubcoreMesh(core_axis_name="c", subcore_axis_name="s")


def _hist_body(ei, deg_out, hist_v, idx_v, red_v, out_v, shared_h):
    c = lax.axis_index("c")
    s = lax.axis_index("s")
    wid = s * 2 + c
    ones16 = jnp.full((16,), 1.0, jnp.float32)
    zeros16 = jnp.zeros((16,), jnp.float32)
    for h in range(6):
        def zero_body(i):
            hist_v[pl.ds(i * 16, 16)] = zeros16
        lax.fori_loop(0, _NPAD // 16, lambda i, _: (zero_body(i), _)[1], None)
        pltpu.sync_copy(ei.at[h, wid], idx_v)

        def acc_body(j):
            idx16 = idx_v[pl.ds(j * 16, 16)]
            plsc.addupdate_scatter(hist_v, [idx16], ones16)
        lax.fori_loop(0, _EH // 16, lambda j, _: (acc_body(j), _)[1], None)

        pltpu.sync_copy(hist_v, shared_h.at[s])
        plsc.subcore_barrier()
        for r in range(16):
            pltpu.sync_copy(shared_h.at[r, pl.ds(s * 640, 640)], red_v.at[r])

        def red_body(j):
            acc = red_v[0, pl.ds(j * 16, 16)]
            for r in range(1, 16):
                acc = acc + red_v[r, pl.ds(j * 16, 16)]
            out_v[pl.ds(j * 16, 16)] = acc
        lax.fori_loop(0, 40, lambda j, _: (red_body(j), _)[1], None)
        pltpu.sync_copy(out_v, deg_out.at[c, h, pl.ds(s * 640, 640)])
        plsc.subcore_barrier()


def _histograms(ei_h):
    k = pl.kernel(
        _hist_body,
        out_type=jax.ShapeDtypeStruct((2, 6, _NPAD), jnp.float32),
        mesh=_sc_mesh(),
        compiler_params=pltpu.CompilerParams(needs_layout_passes=False),
        scratch_types=[
            pltpu.VMEM((_NPAD,), jnp.float32),
            pltpu.VMEM((_EH,), jnp.int32),
            pltpu.VMEM((16, 640), jnp.float32),
            pltpu.VMEM((640,), jnp.float32),
            pltpu.VMEM_SHARED((16, _NPAD), jnp.float32),
        ],
    )
    return k(ei_h)


def _prep_body(dp_ref, sc3_ref):
    deg = dp_ref[0] + dp_ref[1]
    deg_s = jnp.concatenate([deg[0:1], deg[2:3], deg[4:5]], axis=0)
    deg_d = jnp.concatenate([deg[1:2], deg[3:4], deg[5:6]], axis=0)
    sc3_ref[0] = lax.rsqrt(deg_d + 1.0)
    sc3_ref[1] = 1.0 / jnp.maximum(deg_d, 1.0)
    sc3_ref[2] = lax.rsqrt(deg_s + 1.0)


def _prep(deg_part):
    return pl.pallas_call(
        _prep_body,
        out_shape=jax.ShapeDtypeStruct((3, 3, _NPAD), jnp.float32),
    )(deg_part)


def _table_body(x_ref, rs_ref, o_ref):
    t = pl.program_id(0)
    xb = x_ref[0]
    rs = rs_ref[0]
    scale = jnp.where(t == 0, jnp.ones_like(rs), rs)
    o_ref[0, 0] = xb * scale


def _table(xh, rs_s3):
    return pl.pallas_call(
        _table_body,
        grid=(4, 2),
        in_specs=[
            pl.BlockSpec((1, _NPAD, 64), lambda t, c: (c, 0, 0)),
            pl.BlockSpec((1, _NPAD, 1), lambda t, c: (jnp.maximum(t - 1, 0), 0, 0)),
        ],
        out_specs=pl.BlockSpec((1, 1, _NPAD, 64), lambda t, c: (t, c, 0, 0)),
        out_shape=jax.ShapeDtypeStruct((4, 2, _NPAD, 64), jnp.float32),
    )(xh, rs_s3)


_NBUF = 5
_LOOK = 4


def _agg_body(xt, srcr, dstr, acc_out, adj_v, dst_v, rows_v, zero_v,
              acc_sh, *sems):
    gsems, ssems = sems[:_NBUF], sems[_NBUF:]
    c = lax.axis_index("c")
    s = lax.axis_index("s")
    zeros16 = jnp.zeros((16,), jnp.float32)

    def zz_body(i):
        for k in range(64 // 16):
            zero_v[i, pl.ds(k * 16, 16)] = zeros16
    lax.fori_loop(0, _CK, lambda i, _: (zz_body(i), _)[1], None)

    def start_g(g, bi):
        pltpu.async_copy(xt.at[adj_v.at[g]], rows_v.at[bi], gsems[bi])

    def wait_g(g, bi):
        pltpu.make_async_copy(xt.at[adj_v.at[g]], rows_v.at[bi], gsems[bi]).wait()

    def start_s(g, bi):
        pltpu.async_copy(rows_v.at[bi], acc_sh.at[dst_v.at[g]], ssems[bi], add=True)

    def drain_s(g, bi):
        pltpu.make_async_copy(rows_v.at[bi], acc_sh.at[dst_v.at[g]], ssems[bi]).wait()

    def step(g, bi, drain, ahead):
        wait_g(g, bi)
        start_s(g, bi)
        if ahead:
            b2 = (bi + _LOOK) % _NBUF
            if drain:
                drain_s(g, b2)
            start_g(g + _LOOK, b2)

    for b in range(3):
        pltpu.sync_copy(srcr.at[b, s], adj_v)
        pltpu.sync_copy(dstr.at[b, s], dst_v)
        for p in (0, 1):
            for k in range(8):
                pltpu.sync_copy(zero_v, acc_sh.at[pl.ds(s * 640 + k * _CK, _CK)])
            off = c * _NPAD if p == 0 else (1 + b) * 2 * _NPAD

            def adj_body(ch):
                for k in range(_CK // 16):
                    v = adj_v[ch, pl.ds(k * 16, 16)]
                    adj_v[ch, pl.ds(k * 16, 16)] = v + off
            lax.fori_loop(0, _NCH, lambda ch, _: (adj_body(ch), _)[1], None)
            plsc.subcore_barrier()

            for j in range(_LOOK):
                start_g(j, j)
            for bi in range(_NBUF):
                step(bi, bi, drain=(bi >= _NBUF - _LOOK), ahead=True)

            def group(go):
                for bi in range(_NBUF):
                    step(go * _NBUF + bi, bi, True, True)
            lax.fori_loop(1, _NCH // _NBUF - 1,
                          lambda go, _: (group(go), _)[1], None)

            for bi in range(_NBUF):
                g = _NCH - _NBUF + bi
                step(g, bi, True, ahead=(g + _LOOK < _NCH))
            for bi in range(_NBUF):
                drain_s(_NCH - _NBUF + bi, bi)
            plsc.subcore_barrier()

            for k in range(8):
                sl = pl.ds(s * 640 + k * _CK, _CK)
                pltpu.sync_copy(acc_sh.at[sl], acc_out.at[b, p, c, sl])
            plsc.subcore_barrier()


def _aggregate(xt_flat, srcr, dstr):
    k = pl.kernel(
        _agg_body,
        out_type=jax.ShapeDtypeStruct((3, 2, 2, _NPAD, 64), jnp.float32),
        mesh=_sc_mesh(),
        compiler_params=pltpu.CompilerParams(
            needs_layout_passes=False, use_tc_tiling_on_sc=False),
        scratch_types=[
            pltpu.VMEM((_NCH, _CK), jnp.int32),
            pltpu.VMEM((_NCH, _CK), jnp.int32),
            pltpu.VMEM((_NBUF, _CK, 64), jnp.float32),
            pltpu.VMEM((_CK, 64), jnp.float32),
            pltpu.VMEM_SHARED((_NPAD, 64), jnp.float32),
        ] + [pltpu.SemaphoreType.DMA] * (2 * _NBUF),
    )
    return k(xt_flat, srcr, dstr)


def _mix_body(w_ref, x_ref, ns_ref, gs_ref, rsd_ref, inv_ref, W_ref, B_ref, o_ref):
    b = pl.program_id(0)
    ns = jnp.concatenate([ns_ref[0, 0, 0], ns_ref[0, 0, 1]], axis=-1)
    gs = jnp.concatenate([gs_ref[0, 0, 0], gs_ref[0, 0, 1]], axis=-1)
    rsd = rsd_ref[0, 0]
    inv = inv_ref[0, 0]
    x = x_ref[...]
    aggs = (rsd * gs, inv * ns, x + ns)
    acc = None
    for o in range(3):
        h = jnp.dot(aggs[o], W_ref[0, o], preferred_element_type=jnp.float32)
        h = h + B_ref[0, o:o + 1, :]
        e = jnp.where(h > 0, h, jnp.exp(jnp.minimum(h, 0.0)) - 1.0)
        term = w_ref[b, o] * e
        acc = term if acc is None else acc + term
    o_ref[0] = acc


def _mix(weights, x_pad, acc_r, rsd4, inv4, W, B):
    return pl.pallas_call(
        _mix_body,
        grid=(3, _NB),
        in_specs=[
            pl.BlockSpec(memory_space=pltpu.SMEM),
            pl.BlockSpec((_M, _D), lambda b, n: (n, 0)),
            pl.BlockSpec((1, 1, 2, _M, 64), lambda b, n: (b, 0, 0, n, 0)),
            pl.BlockSpec((1, 1, 2, _M, 64), lambda b, n: (b, 1, 0, n, 0)),
            pl.BlockSpec((1, 1, _M, 1), lambda b, n: (b, n, 0, 0)),
            pl.BlockSpec((1, 1, _M, 1), lambda b, n: (b, n, 0, 0)),
            pl.BlockSpec((1, 3, _D, _D), lambda b, n: (b, 0, 0, 0)),
            pl.BlockSpec((1, 3, _D), lambda b, n: (b, 0, 0)),
        ],
        out_specs=pl.BlockSpec((1, _M, _D), lambda b, n: (b, n, 0)),
        out_shape=jax.ShapeDtypeStruct((3, _NPAD, _D), jnp.float32),
    )(weights, x_pad, acc_r, acc_r, rsd4, inv4, W, B)


def kernel(x, weights, edge_index, W, B):
    ei_h = edge_index.reshape(6, _NT, _EH)
    srcr = edge_index[:, 0, :].reshape(3, 16, _NCH, _CK)
    dstr = edge_index[:, 1, :].reshape(3, 16, _NCH, _CK)
    x_pad = jnp.pad(x, ((0, _NPAD - _N), (0, 0)))

    deg_part = _histograms(ei_h)
    sc3 = _prep(deg_part)
    rs_s3 = sc3[2].reshape(3, _NPAD, 1)
    xh = x_pad.reshape(_NPAD, 2, 64).transpose(1, 0, 2)
    xt2 = _table(xh, rs_s3)
    acc_r = _aggregate(xt2.reshape(8 * _NPAD, 64), srcr, dstr)
    rsd4 = sc3[0].reshape(3, _NB, _M, 1)
    inv4 = sc3[1].reshape(3, _NB, _M, 1)
    out = _mix(weights, x_pad, acc_r, rsd4, inv4, W, B)
    return out[:, :_N, :]
